# Initial kernel scaffold; baseline (speedup 1.0000x reference)
import jax
import jax.numpy as jnp
from jax.experimental import pallas as pl


def kernel(label_targs, label_preds, poly_targs, poly_preds, mask_targs, mask_preds, inside_indices):
    raise NotImplementedError("write your pallas kernel here")


# speedup vs baseline: 1.9577x; 1.9577x over previous
import jax
import jax.numpy as jnp
from jax.experimental import pallas as pl
from jax.experimental.pallas import tpu as pltpu

NUM_SAMPLE = 9
ALPHA = 0.25
GAMMA = 2.0
BLK_P = 512


def _cost_topk_kernel(lt_ref, lp_ref, pp_ref, pt_ref, mp_ref, mt_ref,
                      pct_ref, pi_ref, c_ref):
    i = pl.program_id(1)
    nblk = pl.num_programs(1)
    g = lt_ref.shape[2]
    nray = pp_ref.shape[2] // g
    hw = mp_ref.shape[2] // g

    lt = lt_ref[0]
    labels_row = lt[0:1, :]

    x = lp_ref[0]
    lp = jax.nn.sigmoid(x)
    neg = (1.0 - ALPHA) * lp ** GAMMA * -jnp.log(1.0 - lp + 1e-08)
    pos = ALPHA * (1.0 - lp) ** GAMMA * -jnp.log(lp + 1e-08)
    diff = pos - neg
    ncls = x.shape[1]
    onehot = (jax.lax.broadcasted_iota(jnp.int32, (ncls, g), 0)
              == labels_row).astype(jnp.float32)
    cc = jnp.dot(diff, onehot, preferred_element_type=jnp.float32, precision=jax.lax.Precision.HIGHEST)

    ppv = pp_ref[0]
    ptv = pt_ref[0]
    lmax = jnp.maximum(ppv, ptv)
    lmin = jnp.minimum(ppv, ptv)
    segray = (jax.lax.broadcasted_iota(jnp.int32, (g * nray, g), 0) // nray
              == jax.lax.broadcasted_iota(jnp.int32, (g * nray, g), 1)
              ).astype(jnp.float32)
    smax = jnp.dot(lmax, segray, preferred_element_type=jnp.float32, precision=jax.lax.Precision.HIGHEST)
    smin = jnp.dot(lmin, segray, preferred_element_type=jnp.float32, precision=jax.lax.Precision.HIGHEST)
    vm = jnp.log(smax / smin)

    mpv = mp_ref[0]
    mtv = mt_ref[0]
    segm = (jax.lax.broadcasted_iota(jnp.int32, (g * hw, g), 0) // hw
            == jax.lax.broadcasted_iota(jnp.int32, (g * hw, g), 1)
            ).astype(jnp.float32)
    a = jnp.dot(mpv * mtv, segm, preferred_element_type=jnp.float32, precision=jax.lax.Precision.HIGHEST)
    bsum = jnp.dot(mpv, segm, preferred_element_type=jnp.float32, precision=jax.lax.Precision.HIGHEST)
    csum = jnp.dot(mtv, segm, preferred_element_type=jnp.float32, precision=jax.lax.Precision.HIGHEST)
    dice = (2.0 * a + 1.0) / (bsum + csum + 1.0)

    c_ref[pl.ds(i * BLK_P, BLK_P), :] = cc + vm + (1.0 - dice)

    @pl.when(i == nblk - 1)
    def _():
        c = c_ref[:, :]
        p = c.shape[0]
        iota_r = jax.lax.broadcasted_iota(jnp.int32, (p, g), 0)
        iota_c = jax.lax.broadcasted_iota(jnp.int32, (p, g), 1)
        cols8 = jax.lax.broadcasted_iota(jnp.int32, (1, g), 1)
        best = jnp.full((p, g), -1, jnp.int32)
        pi_rows = []
        for k in range(NUM_SAMPLE):
            m = jnp.min(c, axis=0, keepdims=True)
            idxk = jnp.min(jnp.where(c == m, iota_r, p),
                           axis=0, keepdims=True)
            match = iota_r == idxk
            enc = jnp.where(match, (k * g + iota_c) * 128 + labels_row, -1)
            best = jnp.maximum(best, enc)
            pi_rows.append(idxk * g + cols8)
            c = jnp.where(match, jnp.float32(jnp.inf), c)
        best1 = jnp.max(best, axis=1, keepdims=True)
        pct_ref[0] = jnp.where(best1 < 0, 80,
                               jnp.bitwise_and(best1, 127)).astype(jnp.int32)
        pi_rows += [jnp.zeros((1, g), jnp.int32)] * (16 - NUM_SAMPLE)
        pi_ref[0] = jnp.concatenate(pi_rows, axis=0)


def kernel(label_targs, label_preds, poly_targs, poly_preds,
           mask_targs, mask_preds, inside_indices):
    b, p, _ = label_preds.shape
    g = label_targs.shape[1]
    nray = poly_targs.shape[-1]
    hw = mask_targs.shape[-1]
    nblk = p // BLK_P

    lt3 = jnp.broadcast_to(label_targs[:, None, :].astype(jnp.int32),
                           (b, 8, g))
    pp = poly_preds.reshape(b, p, g * nray)
    pt = poly_targs.reshape(b, p, g * nray)
    mp = mask_preds.reshape(b, p, g * hw)
    mt = mask_targs.reshape(b, p, g * hw)

    pct3, pi3 = pl.pallas_call(
        _cost_topk_kernel,
        grid=(b, nblk),
        in_specs=[
            pl.BlockSpec((1, 8, g), lambda bi, i: (bi, 0, 0)),
            pl.BlockSpec((1, BLK_P, 80), lambda bi, i: (bi, i, 0)),
            pl.BlockSpec((1, BLK_P, g * nray), lambda bi, i: (bi, i, 0)),
            pl.BlockSpec((1, BLK_P, g * nray), lambda bi, i: (bi, i, 0)),
            pl.BlockSpec((1, BLK_P, g * hw), lambda bi, i: (bi, i, 0)),
            pl.BlockSpec((1, BLK_P, g * hw), lambda bi, i: (bi, i, 0)),
        ],
        out_specs=[
            pl.BlockSpec((1, p, 1), lambda bi, i: (bi, 0, 0)),
            pl.BlockSpec((1, 16, g), lambda bi, i: (bi, 0, 0)),
        ],
        out_shape=[
            jax.ShapeDtypeStruct((b, p, 1), jnp.int32),
            jax.ShapeDtypeStruct((b, 16, g), jnp.int32),
        ],
        scratch_shapes=[pltpu.VMEM((p, g), jnp.float32)],
        compiler_params=pltpu.CompilerParams(
            dimension_semantics=("arbitrary", "arbitrary")),
    )(lt3, label_preds, pp, pt, mp, mt)

    pos_class_targ = pct3[:, :, 0]
    pos_indices = pi3[:, :NUM_SAMPLE, :].reshape(b, NUM_SAMPLE * g)
    return pos_class_targ, pos_indices


# speedup vs baseline: 2.5355x; 1.2952x over previous
import jax
import jax.numpy as jnp
from jax.experimental import pallas as pl
from jax.experimental.pallas import tpu as pltpu

NUM_SAMPLE = 9
ALPHA = 0.25
GAMMA = 2.0
BLK_P = 512


def _cost_topk_kernel(lt_ref, lp_ref, pp_ref, pt_ref, mp_ref, mt_ref,
                      pct_ref, pi_ref, c_ref):
    i = pl.program_id(1)
    nblk = pl.num_programs(1)
    g = lt_ref.shape[2]
    nray = pp_ref.shape[2] // g
    hw = mp_ref.shape[2] // g

    lt = lt_ref[0]
    labels_row = lt[0:1, :]

    x = lp_ref[0]
    lp = jax.nn.sigmoid(x)
    neg = (1.0 - ALPHA) * lp ** GAMMA * -jnp.log(1.0 - lp + 1e-08)
    pos = ALPHA * (1.0 - lp) ** GAMMA * -jnp.log(lp + 1e-08)
    diff = pos - neg
    ncls = x.shape[1]
    onehot = (jax.lax.broadcasted_iota(jnp.int32, (ncls, g), 0)
              == labels_row).astype(jnp.float32)
    cc = jnp.dot(diff, onehot, preferred_element_type=jnp.float32, precision=jax.lax.Precision.HIGHEST)

    ppv = pp_ref[0]
    ptv = pt_ref[0]
    lmax = jnp.maximum(ppv, ptv)
    lmin = jnp.minimum(ppv, ptv)
    segray = (jax.lax.broadcasted_iota(jnp.int32, (g * nray, g), 0) // nray
              == jax.lax.broadcasted_iota(jnp.int32, (g * nray, g), 1)
              ).astype(jnp.float32)
    smax = jnp.dot(lmax, segray, preferred_element_type=jnp.float32, precision=jax.lax.Precision.HIGHEST)
    smin = jnp.dot(lmin, segray, preferred_element_type=jnp.float32, precision=jax.lax.Precision.HIGHEST)
    vm = jnp.log(smax / smin)

    mpv = mp_ref[0]
    mtv = mt_ref[0]
    hw2 = hw // 2

    def fold(x):
        parts = [x[:, j * hw: j * hw + hw2] + x[:, j * hw + hw2:(j + 1) * hw]
                 for j in range(g)]
        return jnp.concatenate(parts, axis=1)

    prod_f = fold(mpv * mtv)
    sum_f = fold(mpv + mtv)
    segm = (jax.lax.broadcasted_iota(jnp.int32, (g * hw2, g), 0) // hw2
            == jax.lax.broadcasted_iota(jnp.int32, (g * hw2, g), 1)
            ).astype(jnp.float32)
    a = jnp.dot(prod_f, segm, preferred_element_type=jnp.float32, precision=jax.lax.Precision.HIGHEST)
    bc = jnp.dot(sum_f, segm, preferred_element_type=jnp.float32, precision=jax.lax.Precision.HIGHEST)
    dice = (2.0 * a + 1.0) / (bc + 1.0)

    c_ref[pl.ds(i * BLK_P, BLK_P), :] = cc + vm + (1.0 - dice)

    @pl.when(i == nblk - 1)
    def _():
        c = c_ref[:, :]
        p = c.shape[0]
        iota_r = jax.lax.broadcasted_iota(jnp.int32, (p, g), 0)
        iota_c = jax.lax.broadcasted_iota(jnp.int32, (p, g), 1)
        cols8 = jax.lax.broadcasted_iota(jnp.int32, (1, g), 1)
        best = jnp.full((p, g), -1, jnp.int32)
        pi_rows = []
        for k in range(NUM_SAMPLE):
            m = jnp.min(c, axis=0, keepdims=True)
            idxk = jnp.min(jnp.where(c == m, iota_r, p),
                           axis=0, keepdims=True)
            match = iota_r == idxk
            enc = jnp.where(match, (k * g + iota_c) * 128 + labels_row, -1)
            best = jnp.maximum(best, enc)
            pi_rows.append(idxk * g + cols8)
            c = jnp.where(match, jnp.float32(jnp.inf), c)
        best1 = jnp.max(best, axis=1, keepdims=True)
        pct_ref[0] = jnp.where(best1 < 0, 80,
                               jnp.bitwise_and(best1, 127)).astype(jnp.int32)
        pi_rows += [jnp.zeros((1, g), jnp.int32)] * (16 - NUM_SAMPLE)
        pi_ref[0] = jnp.concatenate(pi_rows, axis=0)


def kernel(label_targs, label_preds, poly_targs, poly_preds,
           mask_targs, mask_preds, inside_indices):
    b, p, _ = label_preds.shape
    g = label_targs.shape[1]
    nray = poly_targs.shape[-1]
    hw = mask_targs.shape[-1]
    nblk = p // BLK_P

    lt3 = jnp.broadcast_to(label_targs[:, None, :].astype(jnp.int32),
                           (b, 8, g))
    pp = poly_preds.reshape(b, p, g * nray)
    pt = poly_targs.reshape(b, p, g * nray)
    mp = mask_preds.reshape(b, p, g * hw)
    mt = mask_targs.reshape(b, p, g * hw)

    pct3, pi3 = pl.pallas_call(
        _cost_topk_kernel,
        grid=(b, nblk),
        in_specs=[
            pl.BlockSpec((1, 8, g), lambda bi, i: (bi, 0, 0)),
            pl.BlockSpec((1, BLK_P, 80), lambda bi, i: (bi, i, 0)),
            pl.BlockSpec((1, BLK_P, g * nray), lambda bi, i: (bi, i, 0)),
            pl.BlockSpec((1, BLK_P, g * nray), lambda bi, i: (bi, i, 0)),
            pl.BlockSpec((1, BLK_P, g * hw), lambda bi, i: (bi, i, 0)),
            pl.BlockSpec((1, BLK_P, g * hw), lambda bi, i: (bi, i, 0)),
        ],
        out_specs=[
            pl.BlockSpec((1, p, 1), lambda bi, i: (bi, 0, 0)),
            pl.BlockSpec((1, 16, g), lambda bi, i: (bi, 0, 0)),
        ],
        out_shape=[
            jax.ShapeDtypeStruct((b, p, 1), jnp.int32),
            jax.ShapeDtypeStruct((b, 16, g), jnp.int32),
        ],
        scratch_shapes=[pltpu.VMEM((p, g), jnp.float32)],
        compiler_params=pltpu.CompilerParams(
            dimension_semantics=("arbitrary", "arbitrary")),
    )(lt3, label_preds, pp, pt, mp, mt)

    pos_class_targ = pct3[:, :, 0]
    pos_indices = pi3[:, :NUM_SAMPLE, :].reshape(b, NUM_SAMPLE * g)
    return pos_class_targ, pos_indices


# speedup vs baseline: 3.8263x; 1.5091x over previous
import jax
import jax.numpy as jnp
from jax.experimental import pallas as pl
from jax.experimental.pallas import tpu as pltpu

NUM_SAMPLE = 9
ALPHA = 0.25
GAMMA = 2.0
BLK_P = 512


def _cost_topk_kernel(lt_ref, lp_ref, pp_ref, pt_ref, mp_ref, mt_ref,
                      pct_ref, pi_ref, c_ref):
    i = pl.program_id(1)
    nblk = pl.num_programs(1)
    g = lt_ref.shape[2]
    nray = pp_ref.shape[2]
    hw = mp_ref.shape[2]

    lt = lt_ref[0]
    labels_row = lt[0:1, :]

    x = lp_ref[0]
    lp = jax.nn.sigmoid(x)
    neg = (1.0 - ALPHA) * lp ** GAMMA * -jnp.log(1.0 - lp + 1e-08)
    pos = ALPHA * (1.0 - lp) ** GAMMA * -jnp.log(lp + 1e-08)
    diff = pos - neg
    ncls = x.shape[1]
    onehot = (jax.lax.broadcasted_iota(jnp.int32, (ncls, g), 0)
              == labels_row).astype(jnp.float32)
    cc = jnp.dot(diff, onehot, preferred_element_type=jnp.float32, precision=jax.lax.Precision.HIGHEST)

    ppv = pp_ref[0]
    ptv = pt_ref[0]
    lmax = jnp.maximum(ppv, ptv)
    lmin = jnp.minimum(ppv, ptv)
    ones_r = jnp.ones((nray, 1), jnp.float32)
    smax = jnp.dot(lmax, ones_r, preferred_element_type=jnp.float32, precision=jax.lax.Precision.HIGHEST)
    smin = jnp.dot(lmin, ones_r, preferred_element_type=jnp.float32, precision=jax.lax.Precision.HIGHEST)
    vm_rows = jnp.log(smax / smin)

    mpv = mp_ref[0]
    mtv = mt_ref[0]
    hw2 = hw // 2
    prod = mpv * mtv
    summ = mpv + mtv
    prod_f = prod[:, :hw2] + prod[:, hw2:]
    sum_f = summ[:, :hw2] + summ[:, hw2:]
    ones_h = jnp.ones((hw2, 1), jnp.float32)
    a = jnp.dot(prod_f, ones_h, preferred_element_type=jnp.float32, precision=jax.lax.Precision.HIGHEST)
    bc = jnp.dot(sum_f, ones_h, preferred_element_type=jnp.float32, precision=jax.lax.Precision.HIGHEST)
    dice = (2.0 * a + 1.0) / (bc + 1.0)

    c_rows = vm_rows + (1.0 - dice)
    blk_p = lp_ref.shape[1]
    c_ref[pl.ds(i * blk_p, blk_p), :] = c_rows.reshape(blk_p, g) + cc

    @pl.when(i == nblk - 1)
    def _():
        c = c_ref[:, :]
        p = c.shape[0]
        iota_r = jax.lax.broadcasted_iota(jnp.int32, (p, g), 0)
        iota_c = jax.lax.broadcasted_iota(jnp.int32, (p, g), 1)
        cols8 = jax.lax.broadcasted_iota(jnp.int32, (1, g), 1)
        best = jnp.full((p, g), -1, jnp.int32)
        pi_rows = []
        for k in range(NUM_SAMPLE):
            m = jnp.min(c, axis=0, keepdims=True)
            idxk = jnp.min(jnp.where(c == m, iota_r, p),
                           axis=0, keepdims=True)
            match = iota_r == idxk
            enc = jnp.where(match, (k * g + iota_c) * 128 + labels_row, -1)
            best = jnp.maximum(best, enc)
            pi_rows.append(idxk * g + cols8)
            c = jnp.where(match, jnp.float32(jnp.inf), c)
        best1 = jnp.max(best, axis=1, keepdims=True)
        pct_ref[0] = jnp.where(best1 < 0, 80,
                               jnp.bitwise_and(best1, 127)).astype(jnp.int32)
        pi_rows += [jnp.zeros((1, g), jnp.int32)] * (16 - NUM_SAMPLE)
        pi_ref[0] = jnp.concatenate(pi_rows, axis=0)


def kernel(label_targs, label_preds, poly_targs, poly_preds,
           mask_targs, mask_preds, inside_indices):
    b, p, _ = label_preds.shape
    g = label_targs.shape[1]
    nray = poly_targs.shape[-1]
    hw = mask_targs.shape[-1]
    nblk = p // BLK_P

    lt3 = jnp.broadcast_to(label_targs[:, None, :].astype(jnp.int32),
                           (b, 8, g))
    blk_ni = BLK_P * g

    pct3, pi3 = pl.pallas_call(
        _cost_topk_kernel,
        grid=(b, nblk),
        in_specs=[
            pl.BlockSpec((1, 8, g), lambda bi, i: (bi, 0, 0)),
            pl.BlockSpec((1, BLK_P, 80), lambda bi, i: (bi, i, 0)),
            pl.BlockSpec((1, blk_ni, nray), lambda bi, i: (bi, i, 0)),
            pl.BlockSpec((1, blk_ni, nray), lambda bi, i: (bi, i, 0)),
            pl.BlockSpec((1, blk_ni, hw), lambda bi, i: (bi, i, 0)),
            pl.BlockSpec((1, blk_ni, hw), lambda bi, i: (bi, i, 0)),
        ],
        out_specs=[
            pl.BlockSpec((1, p, 1), lambda bi, i: (bi, 0, 0)),
            pl.BlockSpec((1, 16, g), lambda bi, i: (bi, 0, 0)),
        ],
        out_shape=[
            jax.ShapeDtypeStruct((b, p, 1), jnp.int32),
            jax.ShapeDtypeStruct((b, 16, g), jnp.int32),
        ],
        scratch_shapes=[pltpu.VMEM((p, g), jnp.float32)],
        compiler_params=pltpu.CompilerParams(
            dimension_semantics=("arbitrary", "arbitrary")),
    )(lt3, label_preds, poly_preds, poly_targs, mask_preds, mask_targs)

    pos_class_targ = pct3[:, :, 0]
    pos_indices = pi3[:, :NUM_SAMPLE, :].reshape(b, NUM_SAMPLE * g)
    return pos_class_targ, pos_indices


# speedup vs baseline: 4.6271x; 1.2093x over previous
import jax
import jax.numpy as jnp
from jax.experimental import pallas as pl
from jax.experimental.pallas import tpu as pltpu

NUM_SAMPLE = 9
ALPHA = 0.25
GAMMA = 2.0
BLK_P = 512


def _cost_topk_kernel(lt_ref, lp_ref, pp_ref, pt_ref, mp_ref, mt_ref,
                      pct_ref, pi_ref, c_ref):
    i = pl.program_id(1)
    nblk = pl.num_programs(1)
    g = lt_ref.shape[2]
    nray = pp_ref.shape[2]
    hw = mp_ref.shape[2]

    lt = lt_ref[0]
    labels_row = lt[0:1, :]

    x = lp_ref[0]
    lp = jax.nn.sigmoid(x)
    neg = (1.0 - ALPHA) * lp ** GAMMA * -jnp.log(1.0 - lp + 1e-08)
    pos = ALPHA * (1.0 - lp) ** GAMMA * -jnp.log(lp + 1e-08)
    diff = pos - neg
    ncls = x.shape[1]
    onehot = (jax.lax.broadcasted_iota(jnp.int32, (ncls, g), 0)
              == labels_row).astype(jnp.float32)
    cc = jnp.dot(diff, onehot, preferred_element_type=jnp.float32, precision=jax.lax.Precision.HIGHEST)

    ppv = pp_ref[0]
    ptv = pt_ref[0]
    lmax = jnp.maximum(ppv, ptv)
    lmin = jnp.minimum(ppv, ptv)
    lcat = jnp.concatenate([lmax, lmin], axis=1)
    iot2 = jax.lax.broadcasted_iota(jnp.int32, (2 * nray, 2), 0)
    sel2 = (iot2 // nray
            == jax.lax.broadcasted_iota(jnp.int32, (2 * nray, 2), 1)
            ).astype(jnp.float32)
    smm = jnp.dot(lcat, sel2, preferred_element_type=jnp.float32, precision=jax.lax.Precision.HIGHEST)
    vm_rows = jnp.log(smm[:, 0:1] / smm[:, 1:2])

    mpv = mp_ref[0]
    mtv = mt_ref[0]
    hw2 = hw // 2
    prod = mpv * mtv
    summ = mpv + mtv
    mcat = jnp.concatenate([prod[:, :hw2] + prod[:, hw2:],
                            summ[:, :hw2] + summ[:, hw2:]], axis=1)
    iotm = jax.lax.broadcasted_iota(jnp.int32, (hw, 2), 0)
    selm = (iotm // hw2
            == jax.lax.broadcasted_iota(jnp.int32, (hw, 2), 1)
            ).astype(jnp.float32)
    ab = jnp.dot(mcat, selm, preferred_element_type=jnp.float32, precision=jax.lax.Precision.HIGHEST)
    dice = (2.0 * ab[:, 0:1] + 1.0) / (ab[:, 1:2] + 1.0)

    c_rows = vm_rows + (1.0 - dice)
    blk_p = lp_ref.shape[1]
    c_ref[pl.ds(i * blk_p, blk_p), :] = c_rows.reshape(blk_p, g) + cc

    @pl.when(i == nblk - 1)
    def _():
        c = c_ref[:, :]
        p = c.shape[0]
        iota_r = jax.lax.broadcasted_iota(jnp.int32, (p, g), 0)
        iota_c = jax.lax.broadcasted_iota(jnp.int32, (p, g), 1)
        cols8 = jax.lax.broadcasted_iota(jnp.int32, (1, g), 1)
        best = jnp.full((p, g), -1, jnp.int32)
        pi_rows = []
        for k in range(NUM_SAMPLE):
            m = jnp.min(c, axis=0, keepdims=True)
            idxk = jnp.min(jnp.where(c == m, iota_r, p),
                           axis=0, keepdims=True)
            match = iota_r == idxk
            enc = jnp.where(match, (k * g + iota_c) * 128 + labels_row, -1)
            best = jnp.maximum(best, enc)
            pi_rows.append(idxk * g + cols8)
            c = jnp.where(match, jnp.float32(jnp.inf), c)
        best1 = jnp.max(best, axis=1, keepdims=True)
        pct_ref[0] = jnp.where(best1 < 0, 80,
                               jnp.bitwise_and(best1, 127)).astype(jnp.int32)
        pi_rows += [jnp.zeros((1, g), jnp.int32)] * (16 - NUM_SAMPLE)
        pi_ref[0] = jnp.concatenate(pi_rows, axis=0)


def kernel(label_targs, label_preds, poly_targs, poly_preds,
           mask_targs, mask_preds, inside_indices):
    b, p, _ = label_preds.shape
    g = label_targs.shape[1]
    nray = poly_targs.shape[-1]
    hw = mask_targs.shape[-1]
    nblk = p // BLK_P

    lt3 = jnp.broadcast_to(label_targs[:, None, :].astype(jnp.int32),
                           (b, 8, g))
    blk_ni = BLK_P * g

    pct3, pi3 = pl.pallas_call(
        _cost_topk_kernel,
        grid=(b, nblk),
        in_specs=[
            pl.BlockSpec((1, 8, g), lambda bi, i: (bi, 0, 0)),
            pl.BlockSpec((1, BLK_P, 80), lambda bi, i: (bi, i, 0)),
            pl.BlockSpec((1, blk_ni, nray), lambda bi, i: (bi, i, 0)),
            pl.BlockSpec((1, blk_ni, nray), lambda bi, i: (bi, i, 0)),
            pl.BlockSpec((1, blk_ni, hw), lambda bi, i: (bi, i, 0)),
            pl.BlockSpec((1, blk_ni, hw), lambda bi, i: (bi, i, 0)),
        ],
        out_specs=[
            pl.BlockSpec((1, p, 1), lambda bi, i: (bi, 0, 0)),
            pl.BlockSpec((1, 16, g), lambda bi, i: (bi, 0, 0)),
        ],
        out_shape=[
            jax.ShapeDtypeStruct((b, p, 1), jnp.int32),
            jax.ShapeDtypeStruct((b, 16, g), jnp.int32),
        ],
        scratch_shapes=[pltpu.VMEM((p, g), jnp.float32)],
        compiler_params=pltpu.CompilerParams(
            dimension_semantics=("arbitrary", "arbitrary")),
    )(lt3, label_preds, poly_preds, poly_targs, mask_preds, mask_targs)

    pos_class_targ = pct3[:, :, 0]
    pos_indices = pi3[:, :NUM_SAMPLE, :].reshape(b, NUM_SAMPLE * g)
    return pos_class_targ, pos_indices
